# Initial kernel scaffold; baseline (speedup 1.0000x reference)
import jax
import jax.numpy as jnp
from jax.experimental import pallas as pl


def kernel(h, edge_index, edge_weight, W, a):
    raise NotImplementedError("write your pallas kernel here")


# speedup vs baseline: 14.2588x; 14.2588x over previous
import functools

import jax
import jax.numpy as jnp
from jax import lax
from jax.experimental import pallas as pl
from jax.experimental.pallas import tpu as pltpu
from jax.experimental.pallas import tpu_sc as plsc

N = 10000
E = 320000
D = 128

NC = 2
NS = 16
NW = NC * NS
L = 16

CHUNK = 128
NCHUNK = 79
EPW = CHUNK * NCHUNK
EPAD = EPW * NW
NPAD = 10112
RPT = NPAD // NS
UW = 144
P1W = 16

_f32 = jnp.float32


def _tca_body(h_ref, w_ref, a_ref, wh_ref, s_ref, t_ref):
    wh = jnp.dot(h_ref[...], w_ref[...], preferred_element_type=_f32)
    wh_ref[...] = wh
    s_ref[...] = jnp.sum(wh * a_ref[0:1, :], axis=1, keepdims=True)
    t_ref[...] = jnp.sum(wh * a_ref[1:2, :], axis=1, keepdims=True)


_tca = pl.pallas_call(
    _tca_body,
    out_shape=(
        jax.ShapeDtypeStruct((NPAD, D), _f32),
        jax.ShapeDtypeStruct((NPAD, 1), _f32),
        jax.ShapeDtypeStruct((NPAD, 1), _f32),
    ),
)


def _tcb_body(a1_ref, a2_ref, t_ref, wh_ref, v_ref, e_ref):
    c1 = a1_ref[0] + a1_ref[1]
    c2 = a2_ref[0] + a2_ref[1]
    alpha = c2 + t_ref[...] * c1
    mask = lax.broadcasted_iota(jnp.int32, (NPAD, 1), 0) < N
    neg = jnp.full((NPAD, 1), -jnp.inf, _f32)
    m = jnp.max(jnp.where(mask, alpha, neg))
    aexp = jnp.where(mask, jnp.exp(alpha - m), jnp.zeros((NPAD, 1), _f32))
    v_ref[...] = aexp * wh_ref[...]
    e_ref[...] = aexp


_tcb = pl.pallas_call(
    _tcb_body,
    out_shape=(
        jax.ShapeDtypeStruct((NPAD, D), _f32),
        jax.ShapeDtypeStruct((NPAD, 1), _f32),
    ),
)


def _tcc_body(acc_ref, den_ref, o_ref):
    num = acc_ref[0, 0:N, :] + acc_ref[1, 0:N, :]
    den = den_ref[0, 0:N] + den_ref[1, 0:N]
    x = num / (den + 1e-9)
    o_ref[...] = jnp.where(x > 0, x, jnp.exp(x) - 1.0)


_tcc = pl.pallas_call(
    _tcc_body,
    out_shape=jax.ShapeDtypeStruct((N, D), _f32),
)


def _sc_mesh():
    return plsc.VectorSubcoreMesh(
        core_axis_name="c", subcore_axis_name="s",
        num_cores=NC, num_subcores=NS)


def _sc1_body(s_hbm, src_hbm, dst_hbm, ew_hbm, out1_hbm, out2_hbm,
              src_v, dst_v, ew_v, sval_v, prod_v, zb_v, acc1_sh, acc2_sh,
              sem):
    cid = lax.axis_index("c")
    sid = lax.axis_index("s")
    wid = sid * NC + cid

    pltpu.sync_copy(src_hbm.at[wid], src_v)
    pltpu.sync_copy(dst_hbm.at[wid], dst_v)
    pltpu.sync_copy(ew_hbm.at[wid], ew_v)

    zv = jnp.zeros((L,), _f32)

    @pl.loop(0, (RPT + L - 1) // L)
    def _z(i):
        zb_v[pl.ds(i * L, L)] = zv

    pltpu.sync_copy(zb_v.at[pl.ds(0, RPT)], acc1_sh.at[pl.ds(sid * RPT, RPT)])
    pltpu.sync_copy(zb_v.at[pl.ds(0, RPT)], acc2_sh.at[pl.ds(sid * RPT, RPT)])
    plsc.subcore_barrier()

    @pl.loop(0, NCHUNK)
    def _chunk(j):
        pltpu.async_copy(s_hbm.at[src_v.at[j]], sval_v, sem).wait()
        for i in range(CHUNK // L):
            sl = pl.ds(i * L, L)
            prod_v[sl] = sval_v[sl] * ew_v[j, sl]
        pltpu.sync_copy(ew_v.at[j], acc1_sh.at[dst_v.at[j]], add=True)
        pltpu.sync_copy(prod_v, acc2_sh.at[dst_v.at[j]], add=True)

    plsc.subcore_barrier()
    pltpu.sync_copy(acc1_sh.at[pl.ds(sid * RPT, RPT)], zb_v.at[pl.ds(0, RPT)])
    pltpu.sync_copy(zb_v.at[pl.ds(0, RPT)],
                    out1_hbm.at[pl.ds(cid * NPAD + sid * RPT, RPT)])
    pltpu.sync_copy(acc2_sh.at[pl.ds(sid * RPT, RPT)], zb_v.at[pl.ds(0, RPT)])
    pltpu.sync_copy(zb_v.at[pl.ds(0, RPT)],
                    out2_hbm.at[pl.ds(cid * NPAD + sid * RPT, RPT)])


def _make_sc1():
    return pl.kernel(
        _sc1_body,
        out_type=(
            jax.ShapeDtypeStruct((NC * NPAD,), _f32),
            jax.ShapeDtypeStruct((NC * NPAD,), _f32),
        ),
        mesh=_sc_mesh(),
        scratch_types=[
            pltpu.VMEM((NCHUNK, CHUNK), jnp.int32),
            pltpu.VMEM((NCHUNK, CHUNK), jnp.int32),
            pltpu.VMEM((NCHUNK, CHUNK), _f32),
            pltpu.VMEM((CHUNK,), _f32),
            pltpu.VMEM((CHUNK,), _f32),
            pltpu.VMEM((RPT + 8,), _f32),
            pltpu.VMEM_SHARED((NPAD,), _f32),
            pltpu.VMEM_SHARED((NPAD,), _f32),
            pltpu.SemaphoreType.DMA,
        ],
    )


_WCHUNKS = (CHUNK, CHUNK, CHUNK, CHUNK, RPT - 4 * CHUNK)


def _sc2_body(v_hbm, e_hbm, src_hbm, dst_hbm, out_hbm, den_hbm,
              src_v, dst_v, rows_v, ev_v, zb_v, acc_sh, den_sh, sem, sem2):
    cid = lax.axis_index("c")
    sid = lax.axis_index("s")
    wid = sid * NC + cid

    pltpu.sync_copy(src_hbm.at[wid], src_v)
    pltpu.sync_copy(dst_hbm.at[wid], dst_v)

    zv = jnp.zeros((L,), _f32)

    @pl.loop(0, CHUNK)
    def _z(i):
        for k in range(D // L):
            rows_v[i, pl.ds(k * L, L)] = zv

    @pl.loop(0, (RPT + L - 1) // L)
    def _z2(i):
        zb_v[pl.ds(i * L, L)] = zv

    off = 0
    for w in _WCHUNKS:
        pltpu.sync_copy(rows_v.at[pl.ds(0, w)],
                        acc_sh.at[pl.ds(sid * RPT + off, w)])
        off += w
    pltpu.sync_copy(zb_v.at[pl.ds(0, RPT)], den_sh.at[pl.ds(sid * RPT, RPT)])
    plsc.subcore_barrier()

    @pl.loop(0, NCHUNK)
    def _chunk(j):
        cp1 = pltpu.async_copy(v_hbm.at[src_v.at[j]], rows_v, sem)
        cp2 = pltpu.async_copy(e_hbm.at[src_v.at[j]], ev_v, sem2)
        cp1.wait()
        cp2.wait()
        pltpu.sync_copy(rows_v, acc_sh.at[dst_v.at[j]], add=True)
        pltpu.sync_copy(ev_v, den_sh.at[dst_v.at[j]], add=True)

    plsc.subcore_barrier()
    off = 0
    for w in _WCHUNKS:
        pltpu.sync_copy(acc_sh.at[pl.ds(sid * RPT + off, w)],
                        rows_v.at[pl.ds(0, w)])
        pltpu.sync_copy(rows_v.at[pl.ds(0, w)],
                        out_hbm.at[cid, pl.ds(sid * RPT + off, w)])
        off += w
    pltpu.sync_copy(den_sh.at[pl.ds(sid * RPT, RPT)], zb_v.at[pl.ds(0, RPT)])
    pltpu.sync_copy(zb_v.at[pl.ds(0, RPT)],
                    den_hbm.at[pl.ds(cid * NPAD + sid * RPT, RPT)])


def _make_sc2():
    return pl.kernel(
        _sc2_body,
        out_type=(
            jax.ShapeDtypeStruct((NC, NPAD, D), _f32),
            jax.ShapeDtypeStruct((NC * NPAD,), _f32),
        ),
        mesh=_sc_mesh(),
        scratch_types=[
            pltpu.VMEM((NCHUNK, CHUNK), jnp.int32),
            pltpu.VMEM((NCHUNK, CHUNK), jnp.int32),
            pltpu.VMEM((CHUNK, D), _f32),
            pltpu.VMEM((CHUNK,), _f32),
            pltpu.VMEM((RPT + 8,), _f32),
            pltpu.VMEM_SHARED((NPAD, D), _f32),
            pltpu.VMEM_SHARED((NPAD,), _f32),
            pltpu.SemaphoreType.DMA,
            pltpu.SemaphoreType.DMA,
        ],
    )


@jax.jit
def _run(h, edge_index, edge_weight, W, a):
    src = edge_index[0].astype(jnp.int32)
    dst = edge_index[1].astype(jnp.int32)
    ew = edge_weight.astype(_f32)

    pad_e = EPAD - E
    padi = jnp.full((pad_e,), N, jnp.int32)
    src_r = jnp.concatenate([src, padi]).reshape(NW, NCHUNK, CHUNK)
    dst_r = jnp.concatenate([dst, padi]).reshape(NW, NCHUNK, CHUNK)
    ew_r = jnp.concatenate([ew, jnp.zeros((pad_e,), _f32)]
                           ).reshape(NW, NCHUNK, CHUNK)
    h_p = jnp.concatenate([h, jnp.zeros((NPAD - N, D), _f32)], axis=0)
    a2d = a.reshape(2, D)

    wh, s, t = _tca(h_p, W, a2d)

    c1, c2 = _make_sc1()(s.reshape(NPAD), src_r, dst_r, ew_r)

    v, aexp = _tcb(c1.reshape(NC, NPAD, 1), c2.reshape(NC, NPAD, 1), t, wh)

    acc, den = _make_sc2()(v, aexp.reshape(NPAD), src_r, dst_r)

    return _tcc(acc, den.reshape(NC, NPAD, 1))


def kernel(h, edge_index, edge_weight, W, a):
    return _run(h, edge_index, edge_weight, W, a)
